# baseline (speedup 1.0000x reference)
import functools

import jax
import jax.numpy as jnp
from jax import lax
from jax.experimental import pallas as pl
from jax.experimental.pallas import tpu as pltpu
from jax.experimental.pallas import tpu_sc as plsc

NC, NS, L = 2, 16, 16
NW = NC * NS
GRP = 2


def _build(B, H, D, Dv):
    opw = B // NW
    N = GRP * H
    ng = opw // GRP
    nch = D // L
    mesh = plsc.VectorSubcoreMesh(core_axis_name="c", subcore_axis_name="s")
    dnums = lax.GatherDimensionNumbers(
        offset_dims=(), collapsed_slice_dims=(0,), start_index_map=(0,)
    )

    @functools.partial(
        pl.kernel,
        out_type=jax.ShapeDtypeStruct((B, D), jnp.float32),
        mesh=mesh,
        scratch_types=[
            pltpu.VMEM((ng, N), jnp.int32),
            pltpu.VMEM((N, Dv), jnp.float32),
            pltpu.VMEM((N, Dv), jnp.float32),
            pltpu.VMEM((2 * GRP, D), jnp.float32),
            pltpu.VMEM((1, Dv), jnp.float32),
            pltpu.SemaphoreType.DMA,
            pltpu.SemaphoreType.DMA,
        ],
        compiler_params=pltpu.CompilerParams(use_tc_tiling_on_sc=False),
    )
    def k(trees_hbm, table_hbm, out_hbm, idx_v, bufa, bufb, out_v, t0_v,
          sem0, sem1):
        wid = lax.axis_index("s") * NC + lax.axis_index("c")
        base = wid * opw * H
        for g in range(ng):
            pltpu.sync_copy(trees_hbm.at[pl.ds(base + g * N, N)], idx_v.at[g])
        pltpu.sync_copy(table_hbm.at[pl.ds(0, 1)], t0_v)
        lanes = lax.iota(jnp.int32, L)
        bufs = (bufa, bufb)
        sems = (sem0, sem1)

        def issue(g, slot):
            pltpu.async_copy(table_hbm.at[idx_v.at[g]], bufs[slot], sems[slot])

        def drain(slot):
            pltpu.make_async_copy(
                table_hbm.at[pl.ds(0, N)], bufs[slot], sems[slot]
            ).wait()

        nfull, rem = H // L, H - (H // L) * L
        tm = jnp.minimum(jnp.maximum(lanes - (L - rem - 1), 0), 1)

        def process(g, slot):
            one = jnp.ones((L,), jnp.int32)
            cnts = []
            for j in range(GRP):
                nz = jnp.zeros((L,), jnp.int32)
                for c in range(nfull):
                    v = idx_v[g, pl.ds(j * H + c * L, L)]
                    nz = nz + jnp.minimum(v, one)
                if rem:
                    v = idx_v[g, pl.ds(j * H + H - L, L)]
                    nz = nz + jnp.minimum(v, one) * tm
                for sft in (8, 4, 2, 1):
                    perm = lax.gather(
                        nz, (lanes ^ sft)[:, None], dnums, (1,),
                        mode=lax.GatherScatterMode.PROMISE_IN_BOUNDS,
                    )
                    nz = nz + perm
                cnts.append(jnp.full((L,), H, jnp.int32) - nz)
            drain(slot)

            buf = bufs[slot]
            UR = 8
            for j in range(GRP):
                def acc_body(i, accs, j=j):
                    new = list(accs)
                    for u in range(UR):
                        for c in range(nch):
                            new[c] = new[c] + buf[
                                j * H + i * UR + u, pl.ds(c * L, L)]
                    return tuple(new)

                accs = lax.fori_loop(
                    0, H // UR, acc_body,
                    tuple(jnp.zeros((L,), jnp.float32) for _ in range(nch)),
                )
                cntf = cnts[j].astype(jnp.float32)
                for c in range(nch):
                    out_v[slot * GRP + j, pl.ds(c * L, L)] = (
                        accs[c] - cntf * t0_v[0, pl.ds(c * L, L)]
                    )
            pltpu.sync_copy(
                out_v.at[pl.ds(slot * GRP, GRP)],
                out_hbm.at[pl.ds(wid * opw + g * GRP, GRP)])

        issue(0, 0)

        def grp_body(h, _):
            for p in range(2):
                g = h * 2 + p
                issue(g + 1, 1 - p)
                process(g, p)
            return 0

        lax.fori_loop(0, ng // 2 - 1, grp_body, 0)
        issue(ng - 1, 1)
        process(ng - 2, 0)
        process(ng - 1, 1)

    return k


@jax.jit
def kernel(trees, table):
    B, H = trees.shape
    _, D = table.shape
    t = (trees.astype(jnp.int32) >> 1).reshape(-1)
    table2 = table.reshape(500000, 128)
    return _build(B, H, D, 128)(t, table2)


# speedup vs baseline: 1.1674x; 1.1674x over previous
import functools

import jax
import jax.numpy as jnp
from jax import lax
from jax.experimental import pallas as pl
from jax.experimental.pallas import tpu as pltpu
from jax.experimental.pallas import tpu_sc as plsc

NC, NS, L = 2, 16, 16
NW = NC * NS
GRP = 4


def _build(B, H, D):
    opw = B // NW
    N = GRP * H
    ng = opw // GRP
    nch = D // L
    mesh = plsc.VectorSubcoreMesh(core_axis_name="c", subcore_axis_name="s")
    dnums = lax.GatherDimensionNumbers(
        offset_dims=(), collapsed_slice_dims=(0,), start_index_map=(0,)
    )

    @functools.partial(
        pl.kernel,
        out_type=jax.ShapeDtypeStruct((B, D), jnp.float32),
        mesh=mesh,
        scratch_types=[
            pltpu.VMEM((ng, N), jnp.int32),
            pltpu.VMEM((N, D), jnp.float32),
            pltpu.VMEM((N, D), jnp.float32),
            pltpu.VMEM((2 * GRP, D), jnp.float32),
            pltpu.VMEM((1, D), jnp.float32),
            pltpu.SemaphoreType.DMA,
            pltpu.SemaphoreType.DMA,
        ],
        compiler_params=pltpu.CompilerParams(use_tc_tiling_on_sc=False),
    )
    def k(trees_hbm, table_hbm, out_hbm, idx_v, bufa, bufb, out_v, t0_v,
          sem0, sem1):
        wid = lax.axis_index("s") * NC + lax.axis_index("c")
        base = wid * opw * H
        for g in range(ng):
            pltpu.sync_copy(trees_hbm.at[pl.ds(base + g * N, N)], idx_v.at[g])
        pltpu.sync_copy(table_hbm.at[pl.ds(0, 1)], t0_v)
        lanes = lax.iota(jnp.int32, L)
        bufs = (bufa, bufb)
        sems = (sem0, sem1)

        def issue(g, slot):
            pltpu.async_copy(table_hbm.at[idx_v.at[g]], bufs[slot], sems[slot])

        def drain(slot):
            pltpu.make_async_copy(
                table_hbm.at[pl.ds(0, N)], bufs[slot], sems[slot]
            ).wait()

        nfull, rem = H // L, H - (H // L) * L
        tm = jnp.minimum(jnp.maximum(lanes - (L - rem - 1), 0), 1)

        def process(g, slot):
            one = jnp.ones((L,), jnp.int32)
            cnts = []
            for j in range(GRP):
                nz = jnp.zeros((L,), jnp.int32)
                for c in range(nfull):
                    v = idx_v[g, pl.ds(j * H + c * L, L)]
                    nz = nz + jnp.minimum(v, one)
                if rem:
                    v = idx_v[g, pl.ds(j * H + H - L, L)]
                    nz = nz + jnp.minimum(v, one) * tm
                for sft in (8, 4, 2, 1):
                    perm = lax.gather(
                        nz, (lanes ^ sft)[:, None], dnums, (1,),
                        mode=lax.GatherScatterMode.PROMISE_IN_BOUNDS,
                    )
                    nz = nz + perm
                cnts.append(jnp.full((L,), H, jnp.int32) - nz)
            drain(slot)

            buf = bufs[slot]
            UR = 8
            for j in range(GRP):
                def acc_body(i, accs, j=j):
                    new = list(accs)
                    for u in range(UR):
                        for c in range(nch):
                            new[c] = new[c] + buf[
                                j * H + i * UR + u, pl.ds(c * L, L)]
                    return tuple(new)

                accs = lax.fori_loop(
                    0, H // UR, acc_body,
                    tuple(jnp.zeros((L,), jnp.float32) for _ in range(nch)),
                )
                cntf = cnts[j].astype(jnp.float32)
                for c in range(nch):
                    out_v[slot * GRP + j, pl.ds(c * L, L)] = (
                        accs[c] - cntf * t0_v[0, pl.ds(c * L, L)]
                    )
            pltpu.sync_copy(
                out_v.at[pl.ds(slot * GRP, GRP)],
                out_hbm.at[pl.ds(wid * opw + g * GRP, GRP)])

        issue(0, 0)

        def grp_body(h, _):
            for p in range(2):
                g = h * 2 + p
                issue(g + 1, 1 - p)
                process(g, p)
            return 0

        lax.fori_loop(0, ng // 2 - 1, grp_body, 0)
        issue(ng - 1, 1)
        process(ng - 2, 0)
        process(ng - 1, 1)

    return k


@jax.jit
def kernel(trees, table):
    B, H = trees.shape
    _, D = table.shape
    t = trees.astype(jnp.int32).reshape(-1)
    return _build(B, H, D)(t, table)


# speedup vs baseline: 1.1946x; 1.0233x over previous
import functools

import jax
import jax.numpy as jnp
from jax import lax
from jax.experimental import pallas as pl
from jax.experimental.pallas import tpu as pltpu
from jax.experimental.pallas import tpu_sc as plsc

NC, NS, L = 2, 16, 16
NW = NC * NS
GRP = 4


def _build(B, H, D):
    opw = B // NW
    N = GRP * H
    ng = opw // GRP
    nch = D // L
    mesh = plsc.VectorSubcoreMesh(core_axis_name="c", subcore_axis_name="s")
    dnums = lax.GatherDimensionNumbers(
        offset_dims=(), collapsed_slice_dims=(0,), start_index_map=(0,)
    )

    @functools.partial(
        pl.kernel,
        out_type=jax.ShapeDtypeStruct((B, D), jnp.float32),
        mesh=mesh,
        scratch_types=[
            pltpu.VMEM((ng, N), jnp.int32),
            pltpu.VMEM((N, D), jnp.float32),
            pltpu.VMEM((N, D), jnp.float32),
            pltpu.VMEM((2 * GRP, D), jnp.float32),
            pltpu.VMEM((1, D), jnp.float32),
            pltpu.SemaphoreType.DMA,
            pltpu.SemaphoreType.DMA,
            pltpu.SemaphoreType.DMA,
        ],
        compiler_params=pltpu.CompilerParams(use_tc_tiling_on_sc=False),
    )
    def k(trees_hbm, table_hbm, out_hbm, idx_v, bufa, bufb, out_v, t0_v,
          sem0, sem1, semio):
        wid = lax.axis_index("s") * NC + lax.axis_index("c")
        base = wid * opw * H
        cps = [
            pltpu.async_copy(
                trees_hbm.at[pl.ds(base + g * N, N)], idx_v.at[g], semio)
            for g in range(ng)
        ]
        for cp in cps:
            cp.wait()
        pltpu.sync_copy(table_hbm.at[pl.ds(0, 1)], t0_v)
        lanes = lax.iota(jnp.int32, L)
        bufs = (bufa, bufb)
        sems = (sem0, sem1)

        def issue(g, slot):
            pltpu.async_copy(table_hbm.at[idx_v.at[g]], bufs[slot], sems[slot])

        def drain(slot):
            pltpu.make_async_copy(
                table_hbm.at[pl.ds(0, N)], bufs[slot], sems[slot]
            ).wait()

        nfull, rem = H // L, H - (H // L) * L
        tm = jnp.minimum(jnp.maximum(lanes - (L - rem - 1), 0), 1)

        def process(g, slot):
            one = jnp.ones((L,), jnp.int32)
            cnts = []
            for j in range(GRP):
                nz = jnp.zeros((L,), jnp.int32)
                for c in range(nfull):
                    v = idx_v[g, pl.ds(j * H + c * L, L)]
                    nz = nz + jnp.minimum(v, one)
                if rem:
                    v = idx_v[g, pl.ds(j * H + H - L, L)]
                    nz = nz + jnp.minimum(v, one) * tm
                for sft in (8, 4, 2, 1):
                    perm = lax.gather(
                        nz, (lanes ^ sft)[:, None], dnums, (1,),
                        mode=lax.GatherScatterMode.PROMISE_IN_BOUNDS,
                    )
                    nz = nz + perm
                cnts.append(jnp.full((L,), H, jnp.int32) - nz)
            drain(slot)

            buf = bufs[slot]
            UR = 8
            for j in range(GRP):
                def acc_body(i, accs, j=j):
                    new = list(accs)
                    for u in range(UR):
                        for c in range(nch):
                            new[c] = new[c] + buf[
                                j * H + i * UR + u, pl.ds(c * L, L)]
                    return tuple(new)

                accs = lax.fori_loop(
                    0, H // UR, acc_body,
                    tuple(jnp.zeros((L,), jnp.float32) for _ in range(nch)),
                )
                cntf = cnts[j].astype(jnp.float32)
                for c in range(nch):
                    out_v[slot * GRP + j, pl.ds(c * L, L)] = (
                        accs[c] - cntf * t0_v[0, pl.ds(c * L, L)]
                    )
            pltpu.sync_copy(
                out_v.at[pl.ds(slot * GRP, GRP)],
                out_hbm.at[pl.ds(wid * opw + g * GRP, GRP)])

        issue(0, 0)

        def grp_body(h, _):
            for p in range(2):
                g = h * 2 + p
                issue(g + 1, 1 - p)
                process(g, p)
            return 0

        lax.fori_loop(0, ng // 2 - 1, grp_body, 0)
        issue(ng - 1, 1)
        process(ng - 2, 0)
        process(ng - 1, 1)

    return k


@jax.jit
def kernel(trees, table):
    B, H = trees.shape
    _, D = table.shape
    t = trees.astype(jnp.int32).reshape(-1)
    return _build(B, H, D)(t, table)


# speedup vs baseline: 1.1989x; 1.0036x over previous
import functools

import jax
import jax.numpy as jnp
from jax import lax
from jax.experimental import pallas as pl
from jax.experimental.pallas import tpu as pltpu
from jax.experimental.pallas import tpu_sc as plsc

NC, NS, L = 2, 16, 16
NW = NC * NS
GRP = 4


def _build(B, H, D):
    opw = B // NW
    N = GRP * H
    ng = opw // GRP
    nch = D // L
    mesh = plsc.VectorSubcoreMesh(core_axis_name="c", subcore_axis_name="s")
    dnums = lax.GatherDimensionNumbers(
        offset_dims=(), collapsed_slice_dims=(0,), start_index_map=(0,)
    )

    @functools.partial(
        pl.kernel,
        out_type=jax.ShapeDtypeStruct((B, D), jnp.float32),
        mesh=mesh,
        scratch_types=[
            pltpu.VMEM((ng, N), jnp.int32),
            pltpu.VMEM((N, D), jnp.float32),
            pltpu.VMEM((N, D), jnp.float32),
            pltpu.VMEM((2 * GRP, D), jnp.float32),
            pltpu.VMEM((1, D), jnp.float32),
            pltpu.SemaphoreType.DMA,
            pltpu.SemaphoreType.DMA,
            pltpu.SemaphoreType.DMA,
            pltpu.SemaphoreType.DMA,
            pltpu.SemaphoreType.DMA,
        ],
        compiler_params=pltpu.CompilerParams(use_tc_tiling_on_sc=False),
    )
    def k(trees_hbm, table_hbm, out_hbm, idx_v, bufa, bufb, out_v, t0_v,
          sem0, sem1, semio, semo0, semo1):
        wid = lax.axis_index("s") * NC + lax.axis_index("c")
        base = wid * opw * H
        cps = [
            pltpu.async_copy(
                trees_hbm.at[pl.ds(base + g * N, N)], idx_v.at[g], semio)
            for g in range(ng)
        ]
        for cp in cps:
            cp.wait()
        pltpu.sync_copy(table_hbm.at[pl.ds(0, 1)], t0_v)
        lanes = lax.iota(jnp.int32, L)
        bufs = (bufa, bufb)
        sems = (sem0, sem1)
        semo = (semo0, semo1)
        for slot in range(2):
            pltpu.async_copy(
                out_v.at[pl.ds(slot * GRP, GRP)],
                out_hbm.at[pl.ds(wid * opw, GRP)], semo[slot])

        def out_wait(slot):
            pltpu.make_async_copy(
                out_v.at[pl.ds(slot * GRP, GRP)],
                out_hbm.at[pl.ds(wid * opw, GRP)], semo[slot],
            ).wait()

        def issue(g, slot):
            pltpu.async_copy(table_hbm.at[idx_v.at[g]], bufs[slot], sems[slot])

        def drain(slot):
            pltpu.make_async_copy(
                table_hbm.at[pl.ds(0, N)], bufs[slot], sems[slot]
            ).wait()

        nfull, rem = H // L, H - (H // L) * L
        tm = jnp.minimum(jnp.maximum(lanes - (L - rem - 1), 0), 1)

        def process(g, slot):
            one = jnp.ones((L,), jnp.int32)
            cnts = []
            for j in range(GRP):
                nz = jnp.zeros((L,), jnp.int32)
                for c in range(nfull):
                    v = idx_v[g, pl.ds(j * H + c * L, L)]
                    nz = nz + jnp.minimum(v, one)
                if rem:
                    v = idx_v[g, pl.ds(j * H + H - L, L)]
                    nz = nz + jnp.minimum(v, one) * tm
                for sft in (8, 4, 2, 1):
                    perm = lax.gather(
                        nz, (lanes ^ sft)[:, None], dnums, (1,),
                        mode=lax.GatherScatterMode.PROMISE_IN_BOUNDS,
                    )
                    nz = nz + perm
                cnts.append(jnp.full((L,), H, jnp.int32) - nz)
            drain(slot)
            out_wait(slot)

            buf = bufs[slot]
            UR = 8
            for j in range(GRP):
                def acc_body(i, accs, j=j):
                    new = list(accs)
                    for u in range(UR):
                        for c in range(nch):
                            new[c] = new[c] + buf[
                                j * H + i * UR + u, pl.ds(c * L, L)]
                    return tuple(new)

                accs = lax.fori_loop(
                    0, H // UR, acc_body,
                    tuple(jnp.zeros((L,), jnp.float32) for _ in range(nch)),
                )
                cntf = cnts[j].astype(jnp.float32)
                for c in range(nch):
                    out_v[slot * GRP + j, pl.ds(c * L, L)] = (
                        accs[c] - cntf * t0_v[0, pl.ds(c * L, L)]
                    )
            pltpu.async_copy(
                out_v.at[pl.ds(slot * GRP, GRP)],
                out_hbm.at[pl.ds(wid * opw + g * GRP, GRP)], semo[slot])

        issue(0, 0)

        def grp_body(h, _):
            for p in range(2):
                g = h * 2 + p
                issue(g + 1, 1 - p)
                process(g, p)
            return 0

        lax.fori_loop(0, ng // 2 - 1, grp_body, 0)
        issue(ng - 1, 1)
        process(ng - 2, 0)
        process(ng - 1, 1)
        out_wait(0)
        out_wait(1)

    return k


@jax.jit
def kernel(trees, table):
    B, H = trees.shape
    _, D = table.shape
    t = trees.astype(jnp.int32).reshape(-1)
    return _build(B, H, D)(t, table)
